# Initial kernel scaffold; baseline (speedup 1.0000x reference)
import jax
import jax.numpy as jnp
from jax.experimental import pallas as pl


def kernel(x, edge_index, edge_weight, kernel_1, kernel_2, bias):
    raise NotImplementedError("write your pallas kernel here")


# speedup vs baseline: 4.3932x; 4.3932x over previous
import functools

import jax
import jax.numpy as jnp
from jax import lax
from jax.experimental import pallas as pl
from jax.experimental.pallas import tpu as pltpu
from jax.experimental.pallas import tpu_sc as plsc

N = 10000
E = 320000
D = 128
NW = 32
K = 128
CH = 79
E_PAD = NW * CH * K
N_ACC = 10240
ROWS_PER_SUB = N_ACC // 16
ZR = 128


def _tc_pre_body(x_ref, w1_ref, w2_ref, b_ref, h1_ref, h2_ref):
    x = x_ref[...]
    h1_ref[...] = jnp.dot(x, w1_ref[...], preferred_element_type=jnp.float32) + b_ref[...]
    h2_ref[...] = jnp.dot(x, w2_ref[...], preferred_element_type=jnp.float32)


def _tc_post_body(h1_ref, acc_ref, accw_ref, out_ref):
    s = acc_ref[0, :N, :] + acc_ref[1, :N, :]
    ws = (accw_ref[0, :N] + accw_ref[1, :N])[:, None]
    agg = s / jnp.maximum(ws, 1e-6)
    o = jnp.maximum(h1_ref[...] + agg, 0.0)
    nrm = jnp.sqrt(jnp.sum(o * o, axis=1, keepdims=True))
    out_ref[...] = o / jnp.maximum(nrm, 1e-12)


def _bcast_lane(v16, l):
    idx = jnp.full((16, 1), l, jnp.int32)
    dn = lax.GatherDimensionNumbers(
        offset_dims=(), collapsed_slice_dims=(0,), start_index_map=(0,))
    return lax.gather(v16, idx, dn, (1,),
                      mode=lax.GatherScatterMode.PROMISE_IN_BOUNDS)


def _sc_agg_body(h2_hbm, src_hbm, dst_hbm, w_hbm, out_hbm, outw_hbm,
                 src_v, dst_v, w_v, rows_v, zbuf, wz_v, acc_sh, accw_sh, sem):
    c = lax.axis_index("c")
    s = lax.axis_index("s")

    def zrow(i, carry):
        for j in range(D // 16):
            zbuf[i, pl.ds(j * 16, 16)] = jnp.zeros((16,), jnp.float32)
        return carry
    lax.fori_loop(0, ZR, zrow, 0)

    def zw(i, carry):
        wz_v[pl.ds(i * 16, 16)] = jnp.zeros((16,), jnp.float32)
        return carry
    lax.fori_loop(0, ROWS_PER_SUB // 16, zw, 0)

    for t in range(ROWS_PER_SUB // ZR):
        pltpu.sync_copy(zbuf, acc_sh.at[pl.ds(s * ROWS_PER_SUB + t * ZR, ZR)])
    pltpu.sync_copy(wz_v, accw_sh.at[pl.ds(s * ROWS_PER_SUB, ROWS_PER_SUB)])
    plsc.subcore_barrier()

    wid = s * 2 + c
    base_e = wid * (CH * K)

    def chunk(i, carry):
        eb = base_e + i * K
        pltpu.sync_copy(src_hbm.at[pl.ds(eb, K)], src_v)
        pltpu.sync_copy(dst_hbm.at[pl.ds(eb, K)], dst_v)
        pltpu.sync_copy(w_hbm.at[pl.ds(eb, K)], w_v)
        pltpu.async_copy(h2_hbm.at[src_v], rows_v, sem).wait()

        def group(g, carry2):
            w16 = w_v[pl.ds(g * 16, 16)]
            for l in range(16):
                wb = _bcast_lane(w16, l)
                row = g * 16 + l
                for j in range(D // 16):
                    rows_v[row, pl.ds(j * 16, 16)] = (
                        rows_v[row, pl.ds(j * 16, 16)] * wb)
            return carry2
        lax.fori_loop(0, K // 16, group, 0)

        pltpu.sync_copy(rows_v, acc_sh.at[dst_v], add=True)
        pltpu.sync_copy(w_v, accw_sh.at[dst_v], add=True)
        return carry
    lax.fori_loop(0, CH, chunk, 0)

    plsc.subcore_barrier()
    pltpu.sync_copy(acc_sh.at[pl.ds(s * ROWS_PER_SUB, ROWS_PER_SUB)],
                    out_hbm.at[c, pl.ds(s * ROWS_PER_SUB, ROWS_PER_SUB)])
    pltpu.sync_copy(accw_sh.at[pl.ds(s * ROWS_PER_SUB, ROWS_PER_SUB)], wz_v)
    pltpu.sync_copy(wz_v, outw_hbm.at[pl.ds(c * N_ACC + s * ROWS_PER_SUB,
                                            ROWS_PER_SUB)])


_BR = 1000


def _tc_pre(x, w1, w2, b):
    return pl.pallas_call(
        _tc_pre_body,
        grid=(N // _BR,),
        in_specs=[
            pl.BlockSpec((_BR, D), lambda i: (i, 0)),
            pl.BlockSpec((D, D), lambda i: (0, 0)),
            pl.BlockSpec((D, D), lambda i: (0, 0)),
            pl.BlockSpec((1, D), lambda i: (0, 0)),
        ],
        out_specs=[
            pl.BlockSpec((_BR, D), lambda i: (i, 0)),
            pl.BlockSpec((_BR, D), lambda i: (i, 0)),
        ],
        out_shape=[
            jax.ShapeDtypeStruct((N, D), jnp.float32),
            jax.ShapeDtypeStruct((N, D), jnp.float32),
        ],
    )(x, w1, w2, b)


_sc_agg = functools.partial(
    pl.kernel,
    out_type=[
        jax.ShapeDtypeStruct((2, N_ACC, D), jnp.float32),
        jax.ShapeDtypeStruct((2 * N_ACC,), jnp.float32),
    ],
    mesh=plsc.VectorSubcoreMesh(core_axis_name="c", subcore_axis_name="s"),
    scratch_types=[
        pltpu.VMEM((K,), jnp.int32),
        pltpu.VMEM((K,), jnp.int32),
        pltpu.VMEM((K,), jnp.float32),
        pltpu.VMEM((K, D), jnp.float32),
        pltpu.VMEM((ZR, D), jnp.float32),
        pltpu.VMEM((ROWS_PER_SUB,), jnp.float32),
        pltpu.VMEM_SHARED((N_ACC, D), jnp.float32),
        pltpu.VMEM_SHARED((N_ACC,), jnp.float32),
        pltpu.SemaphoreType.DMA,
    ],
)(_sc_agg_body)


def _tc_post(h1, acc, accw):
    return pl.pallas_call(
        _tc_post_body,
        out_shape=jax.ShapeDtypeStruct((N, D), jnp.float32),
    )(h1, acc, accw)


def kernel(x, edge_index, edge_weight, kernel_1, kernel_2, bias):
    src = edge_index[0].astype(jnp.int32)
    dst = edge_index[1].astype(jnp.int32)
    pad = E_PAD - E
    src = jnp.concatenate([src, jnp.zeros((pad,), jnp.int32)])
    dst = jnp.concatenate([dst, jnp.zeros((pad,), jnp.int32)])
    w = jnp.concatenate([edge_weight.astype(jnp.float32),
                         jnp.zeros((pad,), jnp.float32)])
    h1, h2 = _tc_pre(x, kernel_1, kernel_2, bias.reshape(1, D))
    acc, accw_flat = _sc_agg(h2, src, dst, w)
    return _tc_post(h1, acc, accw_flat.reshape(2, N_ACC))
